# baseline (device time: 318747 ns/iter reference)
import jax
import jax.numpy as jnp
from jax import lax
from jax.experimental import pallas as pl
from jax.experimental.pallas import tpu as pltpu

B, H, D, BS = 32, 16, 128, 32
NB_LOCAL = 256
NB_SLOTS = 256
BPD = B // 4
NSLOT = 4
SCALE = D ** -0.5
NEG = -1e30


def kernel(Q, K, V, bt, lens):
    def body(Q_ref, K_hbm, V_hbm, bt_ref, lens_ref, out_ref,
             o_part, stats, o_rcv, stats_rcv, ag_buf, kbuf, vbuf, comp_s,
             kc_sems, vc_sems, send_sems, recv_sems):
        my_x = lax.axis_index("x")
        my_y = lax.axis_index("y")
        my_z = lax.axis_index("z")
        qid = my_x * 2 + my_z
        b0 = qid * BPD
        lo = (my_y * NB_LOCAL).astype(jnp.int32)

        y_nbr = (my_x, 1 - my_y, my_z)
        z_nbr = (my_x, my_y, 1 - my_z)
        x_nbr = (1 - my_x, my_y, my_z)

        bar = pltpu.get_barrier_semaphore()
        for nbr in (y_nbr, z_nbr, x_nbr):
            pl.semaphore_signal(bar, inc=1, device_id=nbr,
                                device_id_type=pl.DeviceIdType.MESH)
        pl.semaphore_wait(bar, 3)

        def start_copy(slot, page):
            pltpu.make_async_copy(K_hbm.at[page], kbuf.at[slot],
                                  kc_sems.at[slot]).start()
            pltpu.make_async_copy(V_hbm.at[page], vbuf.at[slot],
                                  vc_sems.at[slot]).start()

        def do_batch(bb, carry):
            b = b0 + bb
            q = Q_ref[b, 0]

            def scan_j(jj, cnt):
                local = bt_ref[b, jj] - lo
                v = (local >= 0) & (local < NB_LOCAL)

                @pl.when(v)
                def _():
                    comp_s[0, cnt] = local

                return cnt + jnp.where(v, 1, 0)

            nt = lax.fori_loop(0, lens_ref[b], scan_j, 0)

            for t0 in range(NSLOT - 1):
                @pl.when(nt > t0)
                def _():
                    start_copy(t0, comp_s[0, t0])

            def step(t, mlo):
                m, l, o = mlo
                slot = lax.rem(t, NSLOT)
                pltpu.make_async_copy(K_hbm.at[0], kbuf.at[slot],
                                      kc_sems.at[slot]).wait()
                pltpu.make_async_copy(V_hbm.at[0], vbuf.at[slot],
                                      vc_sems.at[slot]).wait()

                @pl.when(t + NSLOT - 1 < nt)
                def _():
                    start_copy(lax.rem(t + NSLOT - 1, NSLOT),
                               comp_s[0, t + NSLOT - 1])

                kp = kbuf[slot]
                vp = vbuf[slot]
                s = jnp.sum(q[None, :, :] * kp, axis=-1) * SCALE
                m_new = jnp.maximum(m, jnp.max(s, axis=0, keepdims=True))
                corr = jnp.exp(m - m_new)
                p = jnp.exp(s - m_new)
                l_new = l * corr + jnp.sum(p, axis=0, keepdims=True)
                pv = jnp.sum(p[:, :, None] * vp, axis=0)
                o_new = o * corr.reshape(H, 1) + pv
                return (m_new, l_new, o_new)

            init = (jnp.full((1, H), NEG, jnp.float32),
                    jnp.zeros((1, H), jnp.float32),
                    jnp.zeros((H, D), jnp.float32))
            m, l, o = lax.fori_loop(0, nt, step, init)
            stats[0, pl.ds(bb, 1), :] = m
            stats[1, pl.ds(bb, 1), :] = l
            o_part[bb] = o
            return carry

        lax.fori_loop(0, BPD, do_batch, 0)

        r_o = pltpu.make_async_remote_copy(
            src_ref=o_part, dst_ref=o_rcv,
            send_sem=send_sems.at[0], recv_sem=recv_sems.at[0],
            device_id=y_nbr, device_id_type=pl.DeviceIdType.MESH)
        r_s = pltpu.make_async_remote_copy(
            src_ref=stats, dst_ref=stats_rcv,
            send_sem=send_sems.at[1], recv_sem=recv_sems.at[1],
            device_id=y_nbr, device_id_type=pl.DeviceIdType.MESH)
        r_o.start()
        r_s.start()
        r_o.wait()
        r_s.wait()

        m_a = stats[0, :, :]
        l_a = stats[1, :, :]
        m_b = stats_rcv[0, :, :]
        l_b = stats_rcv[1, :, :]
        m = jnp.maximum(m_a, m_b)
        ca = jnp.exp(m_a - m)
        cb = jnp.exp(m_b - m)
        l = l_a * ca + l_b * cb
        o = o_part[...] * ca[:, :, None] + o_rcv[...] * cb[:, :, None]
        ag_buf[pl.ds(b0, BPD)] = o / l[:, :, None]

        r_z = pltpu.make_async_remote_copy(
            src_ref=ag_buf.at[pl.ds(b0, BPD)],
            dst_ref=ag_buf.at[pl.ds(b0, BPD)],
            send_sem=send_sems.at[2], recv_sem=recv_sems.at[2],
            device_id=z_nbr, device_id_type=pl.DeviceIdType.MESH)
        r_z.start()
        r_z.wait()

        x0 = my_x * (2 * BPD)
        r_x = pltpu.make_async_remote_copy(
            src_ref=ag_buf.at[pl.ds(x0, 2 * BPD)],
            dst_ref=ag_buf.at[pl.ds(x0, 2 * BPD)],
            send_sem=send_sems.at[3], recv_sem=recv_sems.at[3],
            device_id=x_nbr, device_id_type=pl.DeviceIdType.MESH)
        r_x.start()
        r_x.wait()

        out_ref[...] = ag_buf[...][:, None, :, :]

    return pl.pallas_call(
        body,
        out_shape=jax.ShapeDtypeStruct((B, 1, H, D), jnp.float32),
        in_specs=[
            pl.BlockSpec(memory_space=pltpu.MemorySpace.VMEM),
            pl.BlockSpec(memory_space=pltpu.MemorySpace.HBM),
            pl.BlockSpec(memory_space=pltpu.MemorySpace.HBM),
            pl.BlockSpec(memory_space=pltpu.MemorySpace.SMEM),
            pl.BlockSpec(memory_space=pltpu.MemorySpace.SMEM),
        ],
        out_specs=pl.BlockSpec(memory_space=pltpu.MemorySpace.VMEM),
        scratch_shapes=[
            pltpu.VMEM((BPD, H, D), jnp.float32),
            pltpu.VMEM((2, BPD, H), jnp.float32),
            pltpu.VMEM((BPD, H, D), jnp.float32),
            pltpu.VMEM((2, BPD, H), jnp.float32),
            pltpu.VMEM((B, H, D), jnp.float32),
            pltpu.VMEM((NSLOT, BS, H, D), jnp.float32),
            pltpu.VMEM((NSLOT, BS, H, D), jnp.float32),
            pltpu.SMEM((1, NB_SLOTS), jnp.int32),
            pltpu.SemaphoreType.DMA((NSLOT,)),
            pltpu.SemaphoreType.DMA((NSLOT,)),
            pltpu.SemaphoreType.DMA((4,)),
            pltpu.SemaphoreType.DMA((4,)),
        ],
        compiler_params=pltpu.CompilerParams(collective_id=0),
    )(Q, K, V, bt, lens)


# device time: 125653 ns/iter; 2.5367x vs baseline; 2.5367x over previous
import jax
import jax.numpy as jnp
from jax import lax
from jax.experimental import pallas as pl
from jax.experimental.pallas import tpu as pltpu

B, H, D, BS = 32, 16, 128, 32
NB_LOCAL = 256
NB_SLOTS = 256
BPD = B // 4
CHUNK = 16
CT = CHUNK * BS
NCHUNK = NB_LOCAL // CHUNK
SCALE = D ** -0.5
NEG = -1e30
BIG = 1e30

NT = (((1,), (1,)), ((), ()))
NN = (((1,), (0,)), ((), ()))


def kernel(Q, K, V, bt, lens):
    def body(Q_ref, K_hbm, V_hbm, bt_ref, lens_ref, out_ref,
             o_part, stats, o_rcv, stats_rcv, ag_buf, kbuf, vbuf,
             m_ref, l_ref, o_ref, M01_ref,
             kc_sems, vc_sems, send_sems, recv_sems):
        my_x = lax.axis_index("x")
        my_y = lax.axis_index("y")
        my_z = lax.axis_index("z")
        qid = my_x * 2 + my_z
        b0 = qid * BPD
        lo = (my_y * NB_LOCAL).astype(jnp.int32)

        y_nbr = (my_x, 1 - my_y, my_z)
        z_nbr = (my_x, my_y, 1 - my_z)
        x_nbr = (1 - my_x, my_y, my_z)

        bar = pltpu.get_barrier_semaphore()
        for nbr in (y_nbr, z_nbr, x_nbr):
            pl.semaphore_signal(bar, inc=1, device_id=nbr,
                                device_id_type=pl.DeviceIdType.MESH)
        pl.semaphore_wait(bar, 3)

        def start_chunk(slot, c):
            src = pl.ds(c * CHUNK, CHUNK)
            pltpu.make_async_copy(K_hbm.at[src], kbuf.at[slot],
                                  kc_sems.at[slot]).start()
            pltpu.make_async_copy(V_hbm.at[src], vbuf.at[slot],
                                  vc_sems.at[slot]).start()

        start_chunk(0, 0)

        pages_row = lax.broadcasted_iota(jnp.int32, (1, NB_LOCAL), 1)
        slot_col = lax.broadcasted_iota(jnp.int32, (NB_SLOTS, 1), 0)
        rows = []
        for b in range(BPD):
            bi = b0 + b
            btb = jnp.transpose(bt_ref[pl.ds(bi, 1), :])
            valid = ((btb - lo) == pages_row) & (slot_col < lens_ref[bi])
            rows.append(jnp.sum(valid.astype(jnp.float32), axis=0,
                                keepdims=True))
        page_mask = jnp.concatenate(rows, axis=0)

        for c in range(NCHUNK):
            pmc = page_mask[:, c * CHUNK:(c + 1) * CHUNK]
            tok = jnp.broadcast_to(pmc[:, :, None], (BPD, CHUNK, BS))
            M01_ref[c] = tok.reshape(BPD, CT)

        m_ref[...] = jnp.full((H, BPD, 1), NEG, jnp.float32)
        l_ref[...] = jnp.zeros((H, BPD, 1), jnp.float32)
        o_ref[...] = jnp.zeros((H, BPD, D), jnp.float32)

        q8 = Q_ref[pl.ds(b0, BPD), 0, :, :]

        def chunk_step(c, carry):
            slot = lax.rem(c, 2)
            pltpu.make_async_copy(K_hbm.at[pl.ds(0, CHUNK)], kbuf.at[slot],
                                  kc_sems.at[slot]).wait()
            pltpu.make_async_copy(V_hbm.at[pl.ds(0, CHUNK)], vbuf.at[slot],
                                  vc_sems.at[slot]).wait()

            @pl.when(c + 1 < NCHUNK)
            def _():
                start_chunk(1 - slot, c + 1)

            mask = M01_ref[c]
            madd = (jnp.minimum(mask, 1.0) - 1.0) * BIG
            for h in range(H):
                qh = q8[:, h, :]
                kp = kbuf[slot][:, :, h, :].reshape(CT, D)
                vp = vbuf[slot][:, :, h, :].reshape(CT, D)
                s = lax.dot_general(
                    qh, kp, NT, preferred_element_type=jnp.float32
                ) * SCALE + madd
                m_old = m_ref[h]
                m_new = jnp.maximum(m_old,
                                    jnp.max(s, axis=1, keepdims=True))
                corr = jnp.exp(m_old - m_new)
                p = jnp.exp(s - m_new) * mask
                l_ref[h] = l_ref[h] * corr + jnp.sum(p, axis=1, keepdims=True)
                m_ref[h] = m_new
                pv = lax.dot_general(
                    p, vp, NN, preferred_element_type=jnp.float32)
                o_ref[h] = o_ref[h] * corr + pv
            return carry

        lax.fori_loop(0, NCHUNK, chunk_step, 0)

        stats[0, :, :] = jnp.transpose(m_ref[...].reshape(H, BPD))
        stats[1, :, :] = jnp.transpose(l_ref[...].reshape(H, BPD))
        o_part[...] = jnp.transpose(o_ref[...], (1, 0, 2))

        r_o = pltpu.make_async_remote_copy(
            src_ref=o_part, dst_ref=o_rcv,
            send_sem=send_sems.at[0], recv_sem=recv_sems.at[0],
            device_id=y_nbr, device_id_type=pl.DeviceIdType.MESH)
        r_s = pltpu.make_async_remote_copy(
            src_ref=stats, dst_ref=stats_rcv,
            send_sem=send_sems.at[1], recv_sem=recv_sems.at[1],
            device_id=y_nbr, device_id_type=pl.DeviceIdType.MESH)
        r_o.start()
        r_s.start()
        r_o.wait()
        r_s.wait()

        m_a = stats[0, :, :]
        l_a = stats[1, :, :]
        m_b = stats_rcv[0, :, :]
        l_b = stats_rcv[1, :, :]
        m = jnp.maximum(m_a, m_b)
        ca = jnp.exp(m_a - m)
        cb = jnp.exp(m_b - m)
        l = l_a * ca + l_b * cb
        o = o_part[...] * ca[:, :, None] + o_rcv[...] * cb[:, :, None]
        ag_buf[pl.ds(b0, BPD)] = o / l[:, :, None]

        r_z = pltpu.make_async_remote_copy(
            src_ref=ag_buf.at[pl.ds(b0, BPD)],
            dst_ref=ag_buf.at[pl.ds(b0, BPD)],
            send_sem=send_sems.at[2], recv_sem=recv_sems.at[2],
            device_id=z_nbr, device_id_type=pl.DeviceIdType.MESH)
        r_z.start()
        r_z.wait()

        x0 = my_x * (2 * BPD)
        r_x = pltpu.make_async_remote_copy(
            src_ref=ag_buf.at[pl.ds(x0, 2 * BPD)],
            dst_ref=ag_buf.at[pl.ds(x0, 2 * BPD)],
            send_sem=send_sems.at[3], recv_sem=recv_sems.at[3],
            device_id=x_nbr, device_id_type=pl.DeviceIdType.MESH)
        r_x.start()
        r_x.wait()

        out_ref[...] = ag_buf[...][:, None, :, :]

    return pl.pallas_call(
        body,
        out_shape=jax.ShapeDtypeStruct((B, 1, H, D), jnp.float32),
        in_specs=[
            pl.BlockSpec(memory_space=pltpu.MemorySpace.VMEM),
            pl.BlockSpec(memory_space=pltpu.MemorySpace.HBM),
            pl.BlockSpec(memory_space=pltpu.MemorySpace.HBM),
            pl.BlockSpec(memory_space=pltpu.MemorySpace.VMEM),
            pl.BlockSpec(memory_space=pltpu.MemorySpace.SMEM),
        ],
        out_specs=pl.BlockSpec(memory_space=pltpu.MemorySpace.VMEM),
        scratch_shapes=[
            pltpu.VMEM((BPD, H, D), jnp.float32),
            pltpu.VMEM((2, BPD, H), jnp.float32),
            pltpu.VMEM((BPD, H, D), jnp.float32),
            pltpu.VMEM((2, BPD, H), jnp.float32),
            pltpu.VMEM((B, H, D), jnp.float32),
            pltpu.VMEM((2, CHUNK, BS, H, D), jnp.float32),
            pltpu.VMEM((2, CHUNK, BS, H, D), jnp.float32),
            pltpu.VMEM((H, BPD, 1), jnp.float32),
            pltpu.VMEM((H, BPD, 1), jnp.float32),
            pltpu.VMEM((H, BPD, D), jnp.float32),
            pltpu.VMEM((NCHUNK, BPD, CT), jnp.float32),
            pltpu.SemaphoreType.DMA((2,)),
            pltpu.SemaphoreType.DMA((2,)),
            pltpu.SemaphoreType.DMA((4,)),
            pltpu.SemaphoreType.DMA((4,)),
        ],
        compiler_params=pltpu.CompilerParams(collective_id=0),
    )(Q, K, V, bt, lens)


# device time: 40142 ns/iter; 7.9405x vs baseline; 3.1302x over previous
import jax
import jax.numpy as jnp
from jax import lax
from jax.experimental import pallas as pl
from jax.experimental.pallas import tpu as pltpu

B, H, D, BS = 32, 16, 128, 32
NB_LOCAL = 256
NB_SLOTS = 256
NB_WIN = 64
CHUNK = 32
CT = CHUNK * BS
NCHUNK = NB_WIN // CHUNK
SCALE = D ** -0.5
NEG = -1e30
BIG = 1e30

NT = (((1,), (1,)), ((), ()))
NN = (((1,), (0,)), ((), ()))


def kernel(Q, K, V, bt, lens):
    def body(Q_ref, K_hbm, V_hbm, bt_ref, lens_ref, out_ref,
             o_acc, stats, o_send_bf, o_rcv, stats_rcv, kbuf, vbuf,
             m_ref, l_ref, o_ref, M01_ref,
             kc_sems, vc_sems, send_sems, recv_sems):
        my_x = lax.axis_index("x")
        my_y = lax.axis_index("y")
        my_z = lax.axis_index("z")
        qid = my_x * 2 + my_z
        p0 = qid * NB_WIN
        lo = (my_y * NB_LOCAL + p0).astype(jnp.int32)

        y_nbr = (my_x, 1 - my_y, my_z)
        z_nbr = (my_x, my_y, 1 - my_z)
        x_nbr = (1 - my_x, my_y, my_z)

        bar = pltpu.get_barrier_semaphore()
        for nbr in (y_nbr, z_nbr, x_nbr):
            pl.semaphore_signal(bar, inc=1, device_id=nbr,
                                device_id_type=pl.DeviceIdType.MESH)
        pl.semaphore_wait(bar, 3)

        def start_chunk(slot, c):
            src = pl.ds(p0 + c * CHUNK, CHUNK)
            for h in range(H):
                pltpu.make_async_copy(K_hbm.at[src, :, h, :],
                                      kbuf.at[slot, h],
                                      kc_sems.at[slot]).start()
                pltpu.make_async_copy(V_hbm.at[src, :, h, :],
                                      vbuf.at[slot, h],
                                      vc_sems.at[slot]).start()

        start_chunk(0, 0)
        start_chunk(1, 1)

        pages_row = lax.broadcasted_iota(jnp.int32, (1, NB_WIN), 1)
        slot_col = lax.broadcasted_iota(jnp.int32, (NB_SLOTS, 1), 0)
        rows = []
        for b in range(B):
            btb = jnp.transpose(bt_ref[pl.ds(b, 1), :])
            valid = ((btb - lo) == pages_row) & (slot_col < lens_ref[b])
            rows.append(jnp.sum(valid.astype(jnp.float32), axis=0,
                                keepdims=True))
        page_mask = jnp.concatenate(rows, axis=0)

        for c in range(NCHUNK):
            pmc = page_mask[:, c * CHUNK:(c + 1) * CHUNK]
            tok = jnp.broadcast_to(pmc[:, :, None], (B, CHUNK, BS))
            M01_ref[c] = tok.reshape(B, CT)

        m_ref[...] = jnp.full((H, B, 1), NEG, jnp.float32)
        l_ref[...] = jnp.zeros((H, B, 1), jnp.float32)
        o_ref[...] = jnp.zeros((H, B, D), jnp.float32)

        q_all = Q_ref[:, 0, :, :]

        def chunk_step(c, carry):
            slot = lax.rem(c, 2)
            for h in range(H):
                pltpu.make_async_copy(K_hbm.at[pl.ds(0, CHUNK), :, h, :],
                                      kbuf.at[slot, h],
                                      kc_sems.at[slot]).wait()
                pltpu.make_async_copy(V_hbm.at[pl.ds(0, CHUNK), :, h, :],
                                      vbuf.at[slot, h],
                                      vc_sems.at[slot]).wait()

            @pl.when(c + 2 < NCHUNK)
            def _():
                start_chunk(lax.rem(c + 2, 2), c + 2)

            mask = M01_ref[c]
            madd = (jnp.minimum(mask, 1.0) - 1.0) * BIG
            for h in range(H):
                qh = q_all[:, h, :]
                kp = kbuf[slot, h].reshape(CT, D)
                vp = vbuf[slot, h].reshape(CT, D)
                s = lax.dot_general(
                    qh, kp, NT, preferred_element_type=jnp.float32
                ) * SCALE + madd
                m_old = m_ref[h]
                m_new = jnp.maximum(m_old,
                                    jnp.max(s, axis=1, keepdims=True))
                corr = jnp.exp(m_old - m_new)
                p = jnp.exp(s - m_new) * mask
                l_ref[h] = l_ref[h] * corr + jnp.sum(p, axis=1, keepdims=True)
                m_ref[h] = m_new
                pv = lax.dot_general(
                    p, vp, NN, preferred_element_type=jnp.float32)
                o_ref[h] = o_ref[h] * corr + pv
            return carry

        lax.fori_loop(0, NCHUNK, chunk_step, 0)

        stats[0, :, :] = jnp.transpose(m_ref[...].reshape(H, B))
        stats[1, :, :] = jnp.transpose(l_ref[...].reshape(H, B))
        o_acc[...] = jnp.transpose(o_ref[...], (1, 0, 2))

        for s, nbr in enumerate((z_nbr, y_nbr, x_nbr)):
            o_send_bf[...] = o_acc[...].astype(jnp.bfloat16)
            r_o = pltpu.make_async_remote_copy(
                src_ref=o_send_bf, dst_ref=o_rcv.at[s],
                send_sem=send_sems.at[2 * s], recv_sem=recv_sems.at[2 * s],
                device_id=nbr, device_id_type=pl.DeviceIdType.MESH)
            r_s = pltpu.make_async_remote_copy(
                src_ref=stats, dst_ref=stats_rcv.at[s],
                send_sem=send_sems.at[2 * s + 1],
                recv_sem=recv_sems.at[2 * s + 1],
                device_id=nbr, device_id_type=pl.DeviceIdType.MESH)
            r_o.start()
            r_s.start()
            r_o.wait()
            r_s.wait()

            m_a = stats[0, :, :]
            l_a = stats[1, :, :]
            m_b = stats_rcv[s, 0, :, :]
            l_b = stats_rcv[s, 1, :, :]
            m = jnp.maximum(m_a, m_b)
            ca = jnp.exp(m_a - m)
            cb = jnp.exp(m_b - m)
            stats[0, :, :] = m
            stats[1, :, :] = l_a * ca + l_b * cb
            o_acc[...] = (o_acc[...] * ca[:, :, None]
                          + o_rcv[s].astype(jnp.float32) * cb[:, :, None])

        out = o_acc[...] / stats[1, :, :][:, :, None]
        out_ref[...] = out[:, None, :, :]

    return pl.pallas_call(
        body,
        out_shape=jax.ShapeDtypeStruct((B, 1, H, D), jnp.float32),
        in_specs=[
            pl.BlockSpec(memory_space=pltpu.MemorySpace.VMEM),
            pl.BlockSpec(memory_space=pltpu.MemorySpace.HBM),
            pl.BlockSpec(memory_space=pltpu.MemorySpace.HBM),
            pl.BlockSpec(memory_space=pltpu.MemorySpace.VMEM),
            pl.BlockSpec(memory_space=pltpu.MemorySpace.SMEM),
        ],
        out_specs=pl.BlockSpec(memory_space=pltpu.MemorySpace.VMEM),
        scratch_shapes=[
            pltpu.VMEM((B, H, D), jnp.float32),
            pltpu.VMEM((2, B, H), jnp.float32),
            pltpu.VMEM((B, H, D), jnp.bfloat16),
            pltpu.VMEM((3, B, H, D), jnp.bfloat16),
            pltpu.VMEM((3, 2, B, H), jnp.float32),
            pltpu.VMEM((2, H, CHUNK, BS, D), jnp.float32),
            pltpu.VMEM((2, H, CHUNK, BS, D), jnp.float32),
            pltpu.VMEM((H, B, 1), jnp.float32),
            pltpu.VMEM((H, B, 1), jnp.float32),
            pltpu.VMEM((H, B, D), jnp.float32),
            pltpu.VMEM((NCHUNK, B, CT), jnp.float32),
            pltpu.SemaphoreType.DMA((3,)),
            pltpu.SemaphoreType.DMA((3,)),
            pltpu.SemaphoreType.DMA((6,)),
            pltpu.SemaphoreType.DMA((6,)),
        ],
        compiler_params=pltpu.CompilerParams(
            collective_id=0, vmem_limit_bytes=100 * 1024 * 1024),
    )(Q, K, V, bt, lens)
